# baseline (device time: 7591 ns/iter reference)
import jax
import jax.numpy as jnp
from jax import lax
from jax.experimental import pallas as pl
from jax.experimental.pallas import tpu as pltpu

N_DEV = 4


def kernel(x, pi):
    def body(pi_ref, x_hbm_ref, out_ref, x_vmem_ref, comm_ref, local_sem,
             send_sem, recv_sem):
        my_pos = lax.axis_index("i")
        dst = pi_ref[my_pos]
        src = sum(
            j * (pi_ref[j] == my_pos).astype(jnp.int32) for j in range(N_DEV)
        )

        barrier_sem = pltpu.get_barrier_semaphore()
        pl.semaphore_signal(
            barrier_sem, inc=1,
            device_id=(src,), device_id_type=pl.DeviceIdType.MESH,
        )
        cp = pltpu.make_async_copy(x_hbm_ref, x_vmem_ref, local_sem)
        cp.start()
        cp.wait()
        comm_ref[0, :, :] = x_vmem_ref[0, :, :].astype(jnp.bfloat16)
        pl.semaphore_wait(barrier_sem, 1)

        rdma = pltpu.make_async_remote_copy(
            src_ref=comm_ref.at[0],
            dst_ref=out_ref.at[0],
            send_sem=send_sem,
            recv_sem=recv_sem,
            device_id=(dst,),
            device_id_type=pl.DeviceIdType.MESH,
        )
        rdma.start()
        rdma.wait()

    return pl.pallas_call(
        body,
        out_shape=jax.ShapeDtypeStruct(x.shape, jnp.bfloat16),
        in_specs=[
            pl.BlockSpec(memory_space=pltpu.SMEM),
            pl.BlockSpec(memory_space=pl.ANY),
        ],
        out_specs=pl.BlockSpec(memory_space=pltpu.VMEM),
        scratch_shapes=[
            pltpu.VMEM((1, 256, 256), jnp.float32),
            pltpu.VMEM((1, 256, 256), jnp.bfloat16),
            pltpu.SemaphoreType.DMA,
            pltpu.SemaphoreType.DMA,
            pltpu.SemaphoreType.DMA,
        ],
        compiler_params=pltpu.CompilerParams(collective_id=0),
    )(pi, x)


# device time: 7537 ns/iter; 1.0072x vs baseline; 1.0072x over previous
import jax
import jax.numpy as jnp
from jax import lax
from jax.experimental import pallas as pl
from jax.experimental.pallas import tpu as pltpu

N_DEV = 4


def kernel(x, pi):
    def body(pi_ref, x_ref, out_ref, comm_ref, send_sem, recv_sem):
        my_pos = lax.axis_index("i")
        dst = pi_ref[my_pos]
        src = sum(
            j * (pi_ref[j] == my_pos).astype(jnp.int32) for j in range(N_DEV)
        )

        barrier_sem = pltpu.get_barrier_semaphore()
        pl.semaphore_signal(
            barrier_sem, inc=1,
            device_id=(src,), device_id_type=pl.DeviceIdType.MESH,
        )
        comm_ref[0, :, :] = x_ref[0, :, :].astype(jnp.bfloat16)
        pl.semaphore_wait(barrier_sem, 1)

        rdma = pltpu.make_async_remote_copy(
            src_ref=comm_ref.at[0],
            dst_ref=out_ref.at[0],
            send_sem=send_sem,
            recv_sem=recv_sem,
            device_id=(dst,),
            device_id_type=pl.DeviceIdType.MESH,
        )
        rdma.start()
        rdma.wait()

    return pl.pallas_call(
        body,
        out_shape=jax.ShapeDtypeStruct(x.shape, jnp.bfloat16),
        in_specs=[
            pl.BlockSpec(memory_space=pltpu.SMEM),
            pl.BlockSpec(memory_space=pltpu.VMEM),
        ],
        out_specs=pl.BlockSpec(memory_space=pltpu.VMEM),
        scratch_shapes=[
            pltpu.VMEM((1, 256, 256), jnp.bfloat16),
            pltpu.SemaphoreType.DMA,
            pltpu.SemaphoreType.DMA,
        ],
        compiler_params=pltpu.CompilerParams(collective_id=0),
    )(pi, x)
